# baseline (speedup 1.0000x reference)
import functools

import jax
import jax.numpy as jnp
import numpy as np
from jax import lax
from jax.experimental import pallas as pl
from jax.experimental.pallas import tpu as pltpu
from jax.experimental.pallas import tpu_sc as plsc

_F = 26
_VD = 1000
_E = 32
_B = 4096
_NC = 2
_NS = 16
_NW = _NC * _NS
_NPAIR = 325
_SLOTS = 11
_TPAD = _SLOTS * _NW + 16
_CHUNK = 2048
_BLK = _VD * _E

_PI, _PJ = np.triu_indices(_F, 1)


def _ffm_body(tbl, lin_tbl, xt32, taskA, taskB, taskFA, taskFB, out,
              task_v, ablk, bblk, linblk, xa_v, xb_v, acc_v, red_v, tmp_v,
              pacc, shared, sem0, sem1):
    cid = lax.axis_index("c")
    sid = lax.axis_index("s")
    wid = sid * _NC + cid
    lane = lax.iota(jnp.int32, 16)
    zero = jnp.zeros((16,), jnp.float32)
    sems = (sem0, sem1)

    pltpu.sync_copy(taskA, task_v.at[0])
    pltpu.sync_copy(taskB, task_v.at[1])
    pltpu.sync_copy(taskFA, task_v.at[2])
    pltpu.sync_copy(taskFB, task_v.at[3])

    def task_scalar(row, task):
        return task_v[row, pl.ds(task, 16)][0]

    def fetch(slot, ch, buf):
        task = slot * _NW + wid
        base_a = pl.multiple_of(task_scalar(0, task), 16)
        base_b = pl.multiple_of(task_scalar(1, task), 16)
        fa = task_scalar(2, task)
        fb = task_scalar(3, task)
        sem = sems[buf]
        cps = [
            pltpu.make_async_copy(tbl.at[pl.ds(base_a, _BLK)], ablk.at[buf],
                                  sem),
            pltpu.make_async_copy(tbl.at[pl.ds(base_b, _BLK)], bblk.at[buf],
                                  sem),
            pltpu.make_async_copy(xt32.at[fa, pl.ds(ch * _CHUNK, _CHUNK)],
                                  xa_v.at[buf], sem),
            pltpu.make_async_copy(xt32.at[fb, pl.ds(ch * _CHUNK, _CHUNK)],
                                  xb_v.at[buf], sem),
        ]
        return cps

    def compute_pair(ch, buf):
        def grp_body(g, _):
            ia = xa_v[buf, pl.ds(g * 16, 16)]
            ib = xb_v[buf, pl.ds(g * 16, 16)]
            for h in range(2):
                oas = [ia[h * 8 + l] for l in range(8)]
                obs = [ib[h * 8 + l] for l in range(8)]
                loads = []
                for l in range(8):
                    loads.append((ablk[buf, pl.ds(oas[l], 32)],
                                  bblk[buf, pl.ds(obs[l], 32)]))
                prods = []
                for va, vb in loads:
                    a0, a1 = plsc.unpack(va, format=plsc.PackFormat.INTERLEAVED)
                    b0, b1 = plsc.unpack(vb, format=plsc.PackFormat.INTERLEAVED)
                    prods.append(a0 * b0 + a1 * b1)
                for l in range(8):
                    plsc.addupdate(
                        pacc.at[pl.ds(g * 272 + (h * 8 + l) * 17, 16)],
                        prods[l])
            return 0

        lax.fori_loop(0, _CHUNK // 16, grp_body, 0)

    for ch in range(2):
        def z_body(i, _):
            pacc[pl.ds(i * 16, 16)] = zero
            return 0

        lax.fori_loop(0, _CHUNK * 17 // 16, z_body, 0)

        pending = None

        @pl.when(wid < _NPAIR)
        def _():
            for cp in fetch(0, ch, 0):
                cp.start()

        for slot in range(_SLOTS):
            buf = slot % 2
            task = slot * _NW + wid
            nxt = (slot + 1) * _NW + wid

            if slot + 1 < _SLOTS:
                @pl.when(nxt < _NPAIR)
                def _():
                    for cp in fetch(slot + 1, ch, 1 - buf):
                        cp.start()

            @pl.when(task < _NPAIR)
            def _():
                for cp in fetch(slot, ch, buf):
                    cp.wait()
                compute_pair(ch, buf)

        def t_body(g, _):
            t0, t1, t2, t3 = zero, zero, zero, zero
            for c in range(16):
                v = plsc.load_gather(pacc, [lane * 17 + (g * 272 + c)])
                if c % 4 == 0:
                    t0 = t0 + v
                elif c % 4 == 1:
                    t1 = t1 + v
                elif c % 4 == 2:
                    t2 = t2 + v
                else:
                    t3 = t3 + v
            acc_v[pl.ds(ch * _CHUNK + g * 16, 16)] = (t0 + t1) + (t2 + t3)
            return 0

        lax.fori_loop(0, _CHUNK // 16, t_body, 0)

    @pl.when(wid < _F)
    def _():
        pltpu.sync_copy(lin_tbl.at[pl.ds(pl.multiple_of(wid * _VD, 8), _VD)],
                        linblk)
        pltpu.sync_copy(xt32.at[wid, pl.ds(0, _CHUNK)], xa_v.at[0])
        pltpu.sync_copy(xt32.at[wid, pl.ds(_CHUNK, _CHUNK)], xb_v.at[0])

        def lin_body(g, _):
            ix = lax.shift_right_logical(xa_v[0, pl.ds(g * 16, 16)], 5)
            acc_v[pl.ds(g * 16, 16)] = (acc_v[pl.ds(g * 16, 16)]
                                        + plsc.load_gather(linblk, [ix]))
            iy = lax.shift_right_logical(xb_v[0, pl.ds(g * 16, 16)], 5)
            acc_v[pl.ds(_CHUNK + g * 16, 16)] = (
                acc_v[pl.ds(_CHUNK + g * 16, 16)]
                + plsc.load_gather(linblk, [iy]))
            return 0

        lax.fori_loop(0, _CHUNK // 16, lin_body, 0)

    pltpu.sync_copy(acc_v, shared.at[sid])
    plsc.subcore_barrier()

    seg = _B // _NS

    def red_zero(i, _):
        red_v[pl.ds(i * 16, 16)] = zero
        return 0

    lax.fori_loop(0, seg // 16, red_zero, 0)

    def red_slot(s, _):
        pltpu.sync_copy(shared.at[s, pl.ds(sid * seg, seg)], tmp_v)

        def red_add(i, _):
            red_v[pl.ds(i * 16, 16)] = (red_v[pl.ds(i * 16, 16)]
                                        + tmp_v[pl.ds(i * 16, 16)])
            return 0

        lax.fori_loop(0, seg // 16, red_add, 0)
        return 0

    lax.fori_loop(0, _NS, red_slot, 0)
    pltpu.sync_copy(red_v, out.at[cid, pl.ds(sid * seg, seg)])


@jax.jit
def _ffm_sc(tbl, lin_tbl, xt32, taskA, taskB, taskFA, taskFB):
    mesh = plsc.VectorSubcoreMesh(core_axis_name="c", subcore_axis_name="s")
    return pl.kernel(
        _ffm_body,
        out_type=jax.ShapeDtypeStruct((_NC, _B), jnp.float32),
        mesh=mesh,
        compiler_params=pltpu.CompilerParams(needs_layout_passes=False,
                                             use_tc_tiling_on_sc=False),
        scratch_types=[
            pltpu.VMEM((4, _TPAD), jnp.int32),
            pltpu.VMEM((2, _BLK), jnp.bfloat16),
            pltpu.VMEM((2, _BLK), jnp.bfloat16),
            pltpu.VMEM((_VD,), jnp.float32),
            pltpu.VMEM((2, _CHUNK), jnp.int32),
            pltpu.VMEM((2, _CHUNK), jnp.int32),
            pltpu.VMEM((_B,), jnp.float32),
            pltpu.VMEM((_B // _NS,), jnp.float32),
            pltpu.VMEM((_B // _NS,), jnp.float32),
            pltpu.VMEM((_CHUNK * 17,), jnp.float32),
            pltpu.VMEM_SHARED((_NS, _B), jnp.float32),
            pltpu.SemaphoreType.DMA,
            pltpu.SemaphoreType.DMA,
        ],
    )(tbl, lin_tbl, xt32, taskA, taskB, taskFA, taskFB)


def kernel(x, W_linear, bias, W_ffm):
    pi = _PI.astype(np.int32)
    pj = _PJ.astype(np.int32)
    base_a = (pj * (_F * _VD) + pi * _VD) * _E
    base_b = (pi * (_F * _VD) + pj * _VD) * _E
    pad = (0, _TPAD - _NPAIR)
    taskA = jnp.asarray(np.pad(base_a, pad), jnp.int32)
    taskB = jnp.asarray(np.pad(base_b, pad), jnp.int32)
    taskFA = jnp.asarray(np.pad(pi, pad), jnp.int32)
    taskFB = jnp.asarray(np.pad(pj, pad), jnp.int32)
    xt32 = (x.T * _E).astype(jnp.int32)
    tbl = W_ffm.astype(jnp.bfloat16).reshape(-1)
    lin_tbl = W_linear.reshape(-1)
    out2 = _ffm_sc(tbl, lin_tbl, xt32, taskA, taskB, taskFA, taskFB)
    return out2[0] + out2[1] + bias[0]


# speedup vs baseline: 1.0048x; 1.0048x over previous
import functools

import jax
import jax.numpy as jnp
import numpy as np
from jax import lax
from jax.experimental import pallas as pl
from jax.experimental.pallas import tpu as pltpu
from jax.experimental.pallas import tpu_sc as plsc

_F = 26
_VD = 1000
_E = 32
_B = 4096
_NC = 2
_NS = 16
_NW = _NC * _NS
_NPAIR = 325
_SLOTS = 11
_TPAD = _SLOTS * _NW + 16
_CHUNK = 2048
_BLK = _VD * _E

_PI, _PJ = np.triu_indices(_F, 1)


def _ffm_body(tbl, lin_tbl, xt32, taskA, out,
              task_v, ablk, bblk, linblk, xa_v, xb_v, acc_v, red_v, tmp_v,
              pacc, shared, sem0, sem1):
    cid = lax.axis_index("c")
    sid = lax.axis_index("s")
    wid = sid * _NC + cid
    lane = lax.iota(jnp.int32, 16)
    zero = jnp.zeros((16,), jnp.float32)
    sems = (sem0, sem1)

    pltpu.sync_copy(taskA, task_v)

    def task_scalar(row, task):
        return task_v[row, pl.ds(task, 16)][0]

    def fetch(slot, ch, buf):
        task = slot * _NW + wid
        ta = task_scalar(0, task)
        off_a = pl.multiple_of(task_scalar(1, task), 16)
        tb = task_scalar(4, task)
        off_b = pl.multiple_of(task_scalar(5, task), 16)
        fa = task_scalar(2, task)
        fb = task_scalar(3, task)
        sem = sems[buf]
        cps = [
            pltpu.make_async_copy(tbl.at[ta, pl.ds(off_a, _BLK)],
                                  ablk.at[buf], sem),
            pltpu.make_async_copy(tbl.at[tb, pl.ds(off_b, _BLK)],
                                  bblk.at[buf], sem),
            pltpu.make_async_copy(xt32.at[fa, pl.ds(ch * _CHUNK, _CHUNK)],
                                  xa_v.at[buf], sem),
            pltpu.make_async_copy(xt32.at[fb, pl.ds(ch * _CHUNK, _CHUNK)],
                                  xb_v.at[buf], sem),
        ]
        return cps

    def compute_pair(ch, buf):
        def grp_body(g, _):
            ia = xa_v[buf, pl.ds(g * 16, 16)]
            ib = xb_v[buf, pl.ds(g * 16, 16)]
            for h in range(2):
                oas = [ia[h * 8 + l] for l in range(8)]
                obs = [ib[h * 8 + l] for l in range(8)]
                loads = []
                for l in range(8):
                    loads.append((ablk[buf, pl.ds(oas[l], 32)],
                                  bblk[buf, pl.ds(obs[l], 32)]))
                prods = []
                for va, vb in loads:
                    a0, a1 = plsc.unpack(va, format=plsc.PackFormat.INTERLEAVED)
                    b0, b1 = plsc.unpack(vb, format=plsc.PackFormat.INTERLEAVED)
                    prods.append(a0 * b0 + a1 * b1)
                for l in range(8):
                    plsc.addupdate(
                        pacc.at[pl.ds(g * 272 + (h * 8 + l) * 17, 16)],
                        prods[l])
            return 0

        lax.fori_loop(0, _CHUNK // 16, grp_body, 0)

    for ch in range(2):
        def z_body(i, _):
            pacc[pl.ds(i * 16, 16)] = zero
            return 0

        lax.fori_loop(0, _CHUNK * 17 // 16, z_body, 0)

        pending = None

        @pl.when(wid < _NPAIR)
        def _():
            for cp in fetch(0, ch, 0):
                cp.start()

        for slot in range(_SLOTS):
            buf = slot % 2
            task = slot * _NW + wid
            nxt = (slot + 1) * _NW + wid

            if slot + 1 < _SLOTS:
                @pl.when(nxt < _NPAIR)
                def _():
                    for cp in fetch(slot + 1, ch, 1 - buf):
                        cp.start()

            @pl.when(task < _NPAIR)
            def _():
                for cp in fetch(slot, ch, buf):
                    cp.wait()
                compute_pair(ch, buf)

        def t_body(g, _):
            t0, t1, t2, t3 = zero, zero, zero, zero
            for c in range(16):
                v = plsc.load_gather(pacc, [lane * 17 + (g * 272 + c)])
                if c % 4 == 0:
                    t0 = t0 + v
                elif c % 4 == 1:
                    t1 = t1 + v
                elif c % 4 == 2:
                    t2 = t2 + v
                else:
                    t3 = t3 + v
            acc_v[pl.ds(ch * _CHUNK + g * 16, 16)] = (t0 + t1) + (t2 + t3)
            return 0

        lax.fori_loop(0, _CHUNK // 16, t_body, 0)

    @pl.when(wid < _F)
    def _():
        pltpu.sync_copy(lin_tbl.at[pl.ds(pl.multiple_of(wid * _VD, 8), _VD)],
                        linblk)
        pltpu.sync_copy(xt32.at[wid, pl.ds(0, _CHUNK)], xa_v.at[0])
        pltpu.sync_copy(xt32.at[wid, pl.ds(_CHUNK, _CHUNK)], xb_v.at[0])

        def lin_body(g, _):
            ix = lax.shift_right_logical(xa_v[0, pl.ds(g * 16, 16)], 5)
            acc_v[pl.ds(g * 16, 16)] = (acc_v[pl.ds(g * 16, 16)]
                                        + plsc.load_gather(linblk, [ix]))
            iy = lax.shift_right_logical(xb_v[0, pl.ds(g * 16, 16)], 5)
            acc_v[pl.ds(_CHUNK + g * 16, 16)] = (
                acc_v[pl.ds(_CHUNK + g * 16, 16)]
                + plsc.load_gather(linblk, [iy]))
            return 0

        lax.fori_loop(0, _CHUNK // 16, lin_body, 0)

    pltpu.sync_copy(acc_v, shared.at[sid])
    plsc.subcore_barrier()

    seg = _B // _NS

    def red_zero(i, _):
        red_v[pl.ds(i * 16, 16)] = zero
        return 0

    lax.fori_loop(0, seg // 16, red_zero, 0)

    def red_slot(s, _):
        pltpu.sync_copy(shared.at[s, pl.ds(sid * seg, seg)], tmp_v)

        def red_add(i, _):
            red_v[pl.ds(i * 16, 16)] = (red_v[pl.ds(i * 16, 16)]
                                        + tmp_v[pl.ds(i * 16, 16)])
            return 0

        lax.fori_loop(0, seg // 16, red_add, 0)
        return 0

    lax.fori_loop(0, _NS, red_slot, 0)
    pltpu.sync_copy(red_v, out.at[cid, pl.ds(sid * seg, seg)])


@jax.jit
def _ffm_sc(tbl, lin_tbl, xt32, taskA):
    mesh = plsc.VectorSubcoreMesh(core_axis_name="c", subcore_axis_name="s")
    return pl.kernel(
        _ffm_body,
        out_type=jax.ShapeDtypeStruct((_NC, _B), jnp.float32),
        mesh=mesh,
        compiler_params=pltpu.CompilerParams(needs_layout_passes=False,
                                             use_tc_tiling_on_sc=False),
        scratch_types=[
            pltpu.VMEM((6, _TPAD), jnp.int32),
            pltpu.VMEM((2, _BLK), jnp.bfloat16),
            pltpu.VMEM((2, _BLK), jnp.bfloat16),
            pltpu.VMEM((_VD,), jnp.float32),
            pltpu.VMEM((2, _CHUNK), jnp.int32),
            pltpu.VMEM((2, _CHUNK), jnp.int32),
            pltpu.VMEM((_B,), jnp.float32),
            pltpu.VMEM((_B // _NS,), jnp.float32),
            pltpu.VMEM((_B // _NS,), jnp.float32),
            pltpu.VMEM((_CHUNK * 17,), jnp.float32),
            pltpu.VMEM_SHARED((_NS, _B), jnp.float32),
            pltpu.SemaphoreType.DMA,
            pltpu.SemaphoreType.DMA,
        ],
    )(tbl, lin_tbl, xt32, taskA)


def kernel(x, W_linear, bias, W_ffm):
    pi = _PI.astype(np.int32)
    pj = _PJ.astype(np.int32)
    pad = (0, _TPAD - _NPAIR)
    taskA = jnp.asarray(np.stack([
        np.pad(pj, pad), np.pad(pi * _VD * _E, pad),
        np.pad(pi, pad), np.pad(pj, pad),
        np.pad(pi, pad), np.pad(pj * _VD * _E, pad),
    ]), jnp.int32)
    xt32 = (x.T * _E).astype(jnp.int32)
    tbl = W_ffm.astype(jnp.bfloat16).reshape(_F, _F * _VD * _E)
    lin_tbl = W_linear.reshape(-1)
    out2 = _ffm_sc(tbl, lin_tbl, xt32, taskA)
    return out2[0] + out2[1] + bias[0]
